# baseline (speedup 1.0000x reference)
import functools

import jax
import jax.numpy as jnp
from jax import lax
from jax.experimental import pallas as pl
from jax.experimental.pallas import tpu as pltpu
from jax.experimental.pallas import tpu_sc as plsc

PAD = 20
NC, NS, LANES = 2, 16, 16
NW = NC * NS

ROWS = 48
PMHC_OFF = 24
CHUNK = 128


def _group_size(L):
    import math
    return LANES // math.gcd(L, LANES)


def _make_hist_call(B, LT, LP):
    SPW = B // NW
    NCHUNK = SPW // CHUNK
    GT = _group_size(LT)
    GP = _group_size(LP)

    mesh = plsc.VectorSubcoreMesh(core_axis_name="c", subcore_axis_name="s")

    @functools.partial(
        pl.kernel,
        out_type=jax.ShapeDtypeStruct((B // CHUNK, ROWS, CHUNK),
                                       jnp.float32),
        mesh=mesh,
        compiler_params=pltpu.CompilerParams(needs_layout_passes=False, use_tc_tiling_on_sc=False),
        scratch_types=[
            pltpu.VMEM((2, CHUNK, LT), jnp.int32),
            pltpu.VMEM((2, CHUNK, LP), jnp.int32),
            pltpu.VMEM((ROWS, CHUNK), jnp.float32),
            pltpu.SemaphoreType.DMA,
            pltpu.SemaphoreType.DMA,
        ],
    )
    def hist_kernel(tcr2_hbm, pmhc2_hbm, out_hbm, tcr_v, pmhc_v, hist_v,
                    sem0, sem1):
        wid = lax.axis_index("s") * NC + lax.axis_index("c")
        base_s = wid * SPW
        iota = lax.iota(jnp.int32, LANES)
        ones = jnp.ones((LANES,), jnp.float32)
        zeros = jnp.zeros((LANES,), jnp.float32)
        sems = (sem0, sem1)

        def start(c):
            par = c % 2
            row0 = base_s + c * CHUNK
            h1 = pltpu.async_copy(
                tcr2_hbm.at[pl.ds(row0, CHUNK)], tcr_v.at[par],
                sems[par])
            h2 = pltpu.async_copy(
                pmhc2_hbm.at[pl.ds(row0, CHUNK)], pmhc_v.at[par],
                sems[par])
            return h1, h2

        pending = start(0)
        for c in range(NCHUNK):
            par = c % 2
            pending[0].wait()
            pending[1].wait()
            if c + 1 < NCHUNK:
                pending = start(c + 1)

            @plsc.parallel_loop(0, ROWS, 1, unroll=2)
            def zero_body(v):
                for j in range(CHUNK // LANES):
                    hist_v[v, pl.ds(j * LANES, LANES)] = zeros

            def stream_body(buf, L, binoff, rot, par=par):
                view = buf.at[par]
                base_cvec = (lax.iota(jnp.int32, LANES) * rot) % L

                for grp in range(CHUNK // LANES):
                    rows = lax.iota(jnp.int32, LANES) + grp * LANES

                    @plsc.parallel_loop(0, L, 1, unroll=4)
                    def pbody(p, rows=rows):
                        cols = jnp.full((LANES,), p, jnp.int32)
                        tok = plsc.load_gather(view, [rows, cols])
                        if binoff:
                            tok = tok + binoff
                        plsc.addupdate_scatter(hist_v, [tok, rows], ones)

            stream_body(tcr_v, LT, 0, 13)
            stream_body(pmhc_v, LP, PMHC_OFF, 1)

            pltpu.sync_copy(hist_v, out_hbm.at[wid * NCHUNK + c])

    return hist_kernel


def _mlp_body(h_ref, wemb_ref, w1_ref, b1_ref, w2_ref, b2_ref, o_ref,
              *, LT, LP, GB):
    for g in range(GB):
        hc = h_ref[g]
        x = jnp.dot(wemb_ref[...], hc, precision=lax.Precision.HIGHEST,
                    preferred_element_type=jnp.float32)
        cnt_t = LT - hc[PAD:PAD + 1, :]
        cnt_p = LP - hc[PMHC_OFF + PAD:PMHC_OFF + PAD + 1, :]
        x = jnp.concatenate([x[0:16, :] / cnt_t, x[16:32, :] / cnt_p], axis=0)
        h = jnp.dot(w1_ref[...], x, precision=lax.Precision.HIGHEST,
                    preferred_element_type=jnp.float32) + b1_ref[...]
        h = jnp.maximum(h, 0.0)
        o = jnp.dot(w2_ref[...], h, precision=lax.Precision.HIGHEST,
                    preferred_element_type=jnp.float32) + b2_ref[...]
        o_ref[g] = o[0]


def _make_mlp_call(B, LT, LP, gb=16):
    nb = B // CHUNK
    return pl.pallas_call(
        functools.partial(_mlp_body, LT=float(LT), LP=float(LP), GB=gb),
        out_shape=jax.ShapeDtypeStruct((nb, CHUNK), jnp.float32),
        grid=(nb // gb,),
        in_specs=[
            pl.BlockSpec((gb, ROWS, CHUNK), lambda i: (i, 0, 0)),
            pl.BlockSpec((32, ROWS), lambda i: (0, 0)),
            pl.BlockSpec((32, 32), lambda i: (0, 0)),
            pl.BlockSpec((32, 1), lambda i: (0, 0)),
            pl.BlockSpec((1, 32), lambda i: (0, 0)),
            pl.BlockSpec((1, 1), lambda i: (0, 0)),
        ],
        out_specs=pl.BlockSpec((gb, CHUNK), lambda i: (i, 0)),
    )


@jax.jit
def kernel(tcr, tcr_len, pmhc, pmhc_len, embed, W1, b1, W2, b2):
    B, LT = tcr.shape
    _, LP = pmhc.shape
    hist = _make_hist_call(B, LT, LP)(
        tcr.astype(jnp.int32), pmhc.astype(jnp.int32))

    et = embed.T.astype(jnp.float32)
    wemb = jnp.zeros((32, ROWS), jnp.float32)
    wemb = wemb.at[0:16, 0:21].set(et)
    wemb = wemb.at[16:32, PMHC_OFF:PMHC_OFF + 21].set(et)

    out = _make_mlp_call(B, LT, LP)(
        hist, wemb, W1.astype(jnp.float32), b1.reshape(32, 1),
        W2.astype(jnp.float32), b2.reshape(1, 1))
    return out.reshape(B)


# speedup vs baseline: 1.1522x; 1.1522x over previous
import functools

import jax
import jax.numpy as jnp
from jax import lax
from jax.experimental import pallas as pl
from jax.experimental.pallas import tpu as pltpu
from jax.experimental.pallas import tpu_sc as plsc

PAD = 20
NC, NS, LANES = 2, 16, 16
NW = NC * NS

ROWS = 48
PMHC_OFF = 24
CHUNK = 128


def _group_size(L):
    import math
    return LANES // math.gcd(L, LANES)


def _make_hist_call(B, LT, LP):
    SPW = B // NW
    NCHUNK = SPW // CHUNK
    GT = _group_size(LT)
    GP = _group_size(LP)

    mesh = plsc.VectorSubcoreMesh(core_axis_name="c", subcore_axis_name="s")

    @functools.partial(
        pl.kernel,
        out_type=jax.ShapeDtypeStruct((ROWS, B), jnp.float32),
        mesh=mesh,
        compiler_params=pltpu.CompilerParams(needs_layout_passes=False, use_tc_tiling_on_sc=False),
        scratch_types=[
            pltpu.VMEM((2, CHUNK, LT), jnp.int32),
            pltpu.VMEM((2, CHUNK, LP), jnp.int32),
            pltpu.VMEM((ROWS, CHUNK), jnp.float32),
            pltpu.SemaphoreType.DMA,
            pltpu.SemaphoreType.DMA,
        ],
    )
    def hist_kernel(tcr2_hbm, pmhc2_hbm, out_hbm, tcr_v, pmhc_v, hist_v,
                    sem0, sem1):
        wid = lax.axis_index("s") * NC + lax.axis_index("c")
        base_s = wid * SPW
        iota = lax.iota(jnp.int32, LANES)
        ones = jnp.ones((LANES,), jnp.float32)
        zeros = jnp.zeros((LANES,), jnp.float32)
        sems = (sem0, sem1)

        def start(c):
            par = c % 2
            row0 = base_s + c * CHUNK
            h1 = pltpu.async_copy(
                tcr2_hbm.at[pl.ds(row0, CHUNK)], tcr_v.at[par],
                sems[par])
            h2 = pltpu.async_copy(
                pmhc2_hbm.at[pl.ds(row0, CHUNK)], pmhc_v.at[par],
                sems[par])
            return h1, h2

        pending = start(0)
        for c in range(NCHUNK):
            par = c % 2
            pending[0].wait()
            pending[1].wait()
            if c + 1 < NCHUNK:
                pending = start(c + 1)

            @plsc.parallel_loop(0, ROWS, 1, unroll=2)
            def zero_body(v):
                for j in range(CHUNK // LANES):
                    hist_v[v, pl.ds(j * LANES, LANES)] = zeros

            def stream_body(buf, L, binoff, rot, par=par):
                view = buf.at[par]
                base_cvec = (lax.iota(jnp.int32, LANES) * rot) % L

                for grp in range(CHUNK // LANES):
                    rows = lax.iota(jnp.int32, LANES) + grp * LANES

                    @plsc.parallel_loop(0, L, 1, unroll=4)
                    def pbody(p, rows=rows):
                        cols = jnp.full((LANES,), p, jnp.int32)
                        tok = plsc.load_gather(view, [rows, cols])
                        if binoff:
                            tok = tok + binoff
                        plsc.addupdate_scatter(hist_v, [tok, rows], ones)

            stream_body(tcr_v, LT, 0, 13)
            stream_body(pmhc_v, LP, PMHC_OFF, 1)

            col0 = base_s + c * CHUNK
            pltpu.sync_copy(hist_v, out_hbm.at[:, pl.ds(col0, CHUNK)])

    return hist_kernel


def _mlp_body(h_ref, wemb_ref, w1_ref, b1_ref, w2_ref, b2_ref, o_ref,
              *, LT, LP):
    hc = h_ref[...]
    x = jnp.dot(wemb_ref[...], hc, precision=lax.Precision.HIGHEST,
                preferred_element_type=jnp.float32)
    cnt_t = LT - hc[PAD:PAD + 1, :]
    cnt_p = LP - hc[PMHC_OFF + PAD:PMHC_OFF + PAD + 1, :]
    x = jnp.concatenate([x[0:16, :] / cnt_t, x[16:32, :] / cnt_p], axis=0)
    h = jnp.dot(w1_ref[...], x, precision=lax.Precision.HIGHEST,
                preferred_element_type=jnp.float32) + b1_ref[...]
    h = jnp.maximum(h, 0.0)
    o = jnp.dot(w2_ref[...], h, precision=lax.Precision.HIGHEST,
                preferred_element_type=jnp.float32) + b2_ref[...]
    o_ref[...] = o


def _make_mlp_call(B, LT, LP, tile=2048):
    return pl.pallas_call(
        functools.partial(_mlp_body, LT=float(LT), LP=float(LP)),
        out_shape=jax.ShapeDtypeStruct((1, B), jnp.float32),
        grid=(B // tile,),
        in_specs=[
            pl.BlockSpec((ROWS, tile), lambda i: (0, i)),
            pl.BlockSpec((32, ROWS), lambda i: (0, 0)),
            pl.BlockSpec((32, 32), lambda i: (0, 0)),
            pl.BlockSpec((32, 1), lambda i: (0, 0)),
            pl.BlockSpec((1, 32), lambda i: (0, 0)),
            pl.BlockSpec((1, 1), lambda i: (0, 0)),
        ],
        out_specs=pl.BlockSpec((1, tile), lambda i: (0, i)),
    )


@jax.jit
def kernel(tcr, tcr_len, pmhc, pmhc_len, embed, W1, b1, W2, b2):
    B, LT = tcr.shape
    _, LP = pmhc.shape
    hist = _make_hist_call(B, LT, LP)(tcr, pmhc)

    et = embed.T.astype(jnp.float32)
    wemb = jnp.zeros((32, ROWS), jnp.float32)
    wemb = wemb.at[0:16, 0:21].set(et)
    wemb = wemb.at[16:32, PMHC_OFF:PMHC_OFF + 21].set(et)

    out = _make_mlp_call(B, LT, LP)(
        hist, wemb, W1.astype(jnp.float32), b1.reshape(32, 1),
        W2.astype(jnp.float32), b2.reshape(1, 1))
    return out[0]
